# baseline (speedup 1.0000x reference)
import functools

import jax
import jax.numpy as jnp
from jax import lax
from jax.experimental import pallas as pl
from jax.experimental.pallas import tpu as pltpu
from jax.experimental.pallas import tpu_sc as plsc

N = 10000
E = 160000
D = 256
H = 512
DH = D // 2

NC = 2
NS = 16
CHUNK = 125
EPT = E // NS
NCHUNK = EPT // CHUNK
HCHUNK = NCHUNK // 2
NROW = N
ROW_STEP = 624
ROW_LEN = 640

_sc_mesh = plsc.VectorSubcoreMesh(core_axis_name="c", subcore_axis_name="s")


@functools.partial(
    pl.kernel,
    out_type=jax.ShapeDtypeStruct((NC, N, DH), jnp.float32),
    mesh=_sc_mesh,
    scratch_types=[
        pltpu.VMEM((HCHUNK, CHUNK), jnp.int32),
        pltpu.VMEM((HCHUNK, CHUNK), jnp.int32),
        pltpu.VMEM((CHUNK, DH), jnp.float32),
        pltpu.VMEM((CHUNK, DH), jnp.float32),
        pltpu.VMEM_SHARED((NROW, DH), jnp.float32),
        pltpu.SemaphoreType.DMA,
        pltpu.SemaphoreType.DMA,
        pltpu.SemaphoreType.DMA,
        pltpu.SemaphoreType.DMA,
    ],
)
def _sc_aggregate(src_hbm, dst_hbm, h2_hbm, zeros_hbm, out_hbm,
                  src_v, dst_v, rows_v0, rows_v1, agg_sh,
                  sem_g0, sem_g1, sem_s0, sem_s1):
    c = lax.axis_index("c")
    s = lax.axis_index("s")
    row0 = s * ROW_STEP

    pltpu.sync_copy(zeros_hbm, agg_sh.at[pl.ds(row0, ROW_LEN)])
    plsc.subcore_barrier()

    def run_half(phase):
        pltpu.sync_copy(src_hbm.at[c, s, pl.ds(phase * HCHUNK, HCHUNK)], src_v)
        pltpu.sync_copy(dst_hbm.at[s, pl.ds(phase * HCHUNK, HCHUNK)], dst_v)
        def gather(j, buf, sem):
            pltpu.async_copy(h2_hbm.at[src_v.at[j]], buf, sem)

        def wait(buf, sem):
            pltpu.make_async_copy(h2_hbm.at[src_v.at[0]], buf, sem).wait()

        gather(0, rows_v0, sem_g0)
        gather(1, rows_v1, sem_g1)

        def u_body(u, carry):
            j0 = 2 * u
            j1 = j0 + 1
            wait(rows_v0, sem_g0)
            pltpu.async_copy(rows_v0, agg_sh.at[dst_v.at[j0]], sem_s0,
                             add=True)
            wait(rows_v1, sem_g1)
            pltpu.async_copy(rows_v1, agg_sh.at[dst_v.at[j1]], sem_s1,
                             add=True)
            wait(rows_v0, sem_s0)
            gather(lax.min(j0 + 2, HCHUNK - 1), rows_v0, sem_g0)
            wait(rows_v1, sem_s1)
            gather(lax.min(j1 + 2, HCHUNK - 1), rows_v1, sem_g1)
            return carry

        lax.fori_loop(0, HCHUNK // 2, u_body, 0)
        wait(rows_v0, sem_g0)
        wait(rows_v1, sem_g1)

    run_half(0)
    run_half(1)
    plsc.subcore_barrier()

    pltpu.sync_copy(agg_sh.at[pl.ds(row0, ROW_LEN)],
                    out_hbm.at[c, pl.ds(row0, ROW_LEN)])


_ROW_BLK = 1000


def _tc_matmul_body(a0_ref, a1_ref, w_ref, b_ref, o_ref):
    w = w_ref[...]
    acc = lax.dot_general(a0_ref[0], w[:, :DH],
                          (((1,), (1,)), ((), ())),
                          preferred_element_type=jnp.float32)
    acc += lax.dot_general(a1_ref[0], w[:, DH:],
                           (((1,), (1,)), ((), ())),
                           preferred_element_type=jnp.float32)
    o_ref[...] = acc + b_ref[...]


@jax.jit
def kernel(edge_index, h, W, b):
    src = edge_index[0]
    dst = edge_index[1]
    src_t = src.reshape(NS, NCHUNK, CHUNK)
    src2 = jnp.stack([2 * src_t, 2 * src_t + 1])
    dst_t = dst.reshape(NS, NCHUNK, CHUNK)
    h2 = h.reshape(2 * N, DH)
    zeros = jnp.zeros((ROW_LEN, DH), jnp.float32)

    agg2 = _sc_aggregate(src2, dst_t, h2, zeros)

    out = pl.pallas_call(
        _tc_matmul_body,
        grid=(N // _ROW_BLK,),
        in_specs=[
            pl.BlockSpec((1, _ROW_BLK, DH), lambda i: (0, i, 0)),
            pl.BlockSpec((1, _ROW_BLK, DH), lambda i: (1, i, 0)),
            pl.BlockSpec((H, D), lambda i: (0, 0)),
            pl.BlockSpec((1, H), lambda i: (0, 0)),
        ],
        out_specs=pl.BlockSpec((_ROW_BLK, H), lambda i: (i, 0)),
        out_shape=jax.ShapeDtypeStruct((N, H), jnp.float32),
    )(agg2, agg2, W, b.reshape(1, H))
    return out


# speedup vs baseline: 1.2508x; 1.2508x over previous
import functools

import jax
import jax.numpy as jnp
from jax import lax
from jax.experimental import pallas as pl
from jax.experimental.pallas import tpu as pltpu
from jax.experimental.pallas import tpu_sc as plsc

N = 10000
E = 160000
D = 256
H = 512
DH = D // 2

NC = 2
NS = 16
CHUNK = 125
EPT = E // NS
NCHUNK = EPT // CHUNK
HCHUNK = NCHUNK // 2
NROW = N
ROW_STEP = 624
ROW_LEN = 640

_sc_mesh = plsc.VectorSubcoreMesh(core_axis_name="c", subcore_axis_name="s")


@functools.partial(
    pl.kernel,
    out_type=jax.ShapeDtypeStruct((NC, N, DH), jnp.float32),
    mesh=_sc_mesh,
    scratch_types=[
        pltpu.VMEM((HCHUNK, CHUNK), jnp.int32),
        pltpu.VMEM((HCHUNK, CHUNK), jnp.int32),
        pltpu.VMEM((CHUNK, DH), jnp.float32),
        pltpu.VMEM((CHUNK, DH), jnp.float32),
        pltpu.VMEM_SHARED((NROW, DH), jnp.float32),
        pltpu.SemaphoreType.DMA,
        pltpu.SemaphoreType.DMA,
    ],
)
def _sc_aggregate(src_hbm, dst_hbm, h2_hbm, zeros_hbm, out_hbm,
                  src_v, dst_v, rows_v0, rows_v1, agg_sh, sem0, sem1):
    c = lax.axis_index("c")
    s = lax.axis_index("s")
    row0 = s * ROW_STEP

    pltpu.sync_copy(zeros_hbm, agg_sh.at[pl.ds(row0, ROW_LEN)])
    plsc.subcore_barrier()

    def run_half(phase):
        pltpu.sync_copy(src_hbm.at[c, s, pl.ds(phase * HCHUNK, HCHUNK)], src_v)
        pltpu.sync_copy(dst_hbm.at[s, pl.ds(phase * HCHUNK, HCHUNK)], dst_v)
        pltpu.async_copy(h2_hbm.at[src_v.at[0]], rows_v0, sem0)

        def u_body(u, carry):
            j0 = 2 * u
            j1 = j0 + 1
            pltpu.async_copy(h2_hbm.at[src_v.at[j1]], rows_v1, sem1)
            pltpu.make_async_copy(h2_hbm.at[src_v.at[j0]], rows_v0, sem0).wait()
            pltpu.sync_copy(rows_v0, agg_sh.at[dst_v.at[j0]], add=True)
            jn = lax.min(j0 + 2, HCHUNK - 1)
            pltpu.async_copy(h2_hbm.at[src_v.at[jn]], rows_v0, sem0)
            pltpu.make_async_copy(h2_hbm.at[src_v.at[j1]], rows_v1, sem1).wait()
            pltpu.sync_copy(rows_v1, agg_sh.at[dst_v.at[j1]], add=True)
            return carry

        lax.fori_loop(0, HCHUNK // 2, u_body, 0)
        pltpu.make_async_copy(h2_hbm.at[src_v.at[0]], rows_v0, sem0).wait()

    run_half(0)
    run_half(1)
    plsc.subcore_barrier()

    pltpu.sync_copy(agg_sh.at[pl.ds(row0, ROW_LEN)],
                    out_hbm.at[c, pl.ds(row0, ROW_LEN)])


_ROW_BLK = 2000


def _tc_matmul_body(a0_ref, a1_ref, w_ref, b_ref, o_ref):
    w = w_ref[...]
    acc = lax.dot_general(a0_ref[0], w[:, :DH],
                          (((1,), (1,)), ((), ())),
                          preferred_element_type=jnp.float32)
    acc += lax.dot_general(a1_ref[0], w[:, DH:],
                           (((1,), (1,)), ((), ())),
                           preferred_element_type=jnp.float32)
    o_ref[...] = acc + b_ref[...]


@jax.jit
def kernel(edge_index, h, W, b):
    src = edge_index[0]
    dst = edge_index[1]
    src_t = src.reshape(NS, NCHUNK, CHUNK)
    src2 = jnp.stack([2 * src_t, 2 * src_t + 1])
    dst_t = dst.reshape(NS, NCHUNK, CHUNK)
    h2 = h.reshape(2 * N, DH)
    zeros = jnp.zeros((ROW_LEN, DH), jnp.float32)

    agg2 = _sc_aggregate(src2, dst_t, h2, zeros)

    out = pl.pallas_call(
        _tc_matmul_body,
        grid=(N // _ROW_BLK,),
        in_specs=[
            pl.BlockSpec((1, _ROW_BLK, DH), lambda i: (0, i, 0)),
            pl.BlockSpec((1, _ROW_BLK, DH), lambda i: (1, i, 0)),
            pl.BlockSpec((H, D), lambda i: (0, 0)),
            pl.BlockSpec((1, H), lambda i: (0, 0)),
        ],
        out_specs=pl.BlockSpec((_ROW_BLK, H), lambda i: (i, 0)),
        out_shape=jax.ShapeDtypeStruct((N, H), jnp.float32),
    )(agg2, agg2, W, b.reshape(1, H))
    return out


# speedup vs baseline: 1.2535x; 1.0022x over previous
import functools

import jax
import jax.numpy as jnp
from jax import lax
from jax.experimental import pallas as pl
from jax.experimental.pallas import tpu as pltpu
from jax.experimental.pallas import tpu_sc as plsc

N = 10000
E = 160000
D = 256
H = 512
DH = D // 2

NC = 2
NS = 16
CHUNK = 125
EPT = E // NS
NCHUNK = EPT // CHUNK
HCHUNK = NCHUNK // 2
NROW = N
ROW_STEP = 624
ROW_LEN = 640

_sc_mesh = plsc.VectorSubcoreMesh(core_axis_name="c", subcore_axis_name="s")


@functools.partial(
    pl.kernel,
    out_type=jax.ShapeDtypeStruct((NC, N, DH), jnp.float32),
    mesh=_sc_mesh,
    scratch_types=[
        pltpu.VMEM((HCHUNK, CHUNK), jnp.int32),
        pltpu.VMEM((HCHUNK, CHUNK), jnp.int32),
        pltpu.VMEM((CHUNK, DH), jnp.float32),
        pltpu.VMEM((CHUNK, DH), jnp.float32),
        pltpu.VMEM_SHARED((NROW, DH), jnp.float32),
        pltpu.SemaphoreType.DMA,
        pltpu.SemaphoreType.DMA,
    ],
)
def _sc_aggregate(src_hbm, dst_hbm, h2_hbm, zeros_hbm, out_hbm,
                  src_v, dst_v, rows_v0, rows_v1, agg_sh, sem0, sem1):
    c = lax.axis_index("c")
    s = lax.axis_index("s")
    row0 = s * ROW_STEP

    def stage_idx(phase):
        pltpu.sync_copy(src_hbm.at[c, s, pl.ds(phase * HCHUNK, HCHUNK)], src_v)
        pltpu.sync_copy(dst_hbm.at[s, pl.ds(phase * HCHUNK, HCHUNK)], dst_v)

    stage_idx(0)
    pltpu.sync_copy(zeros_hbm, agg_sh.at[pl.ds(row0, ROW_LEN)])
    plsc.subcore_barrier()

    def run_half(phase):
        if phase:
            stage_idx(phase)
        pltpu.async_copy(h2_hbm.at[src_v.at[0]], rows_v0, sem0)

        def u_body(u, carry):
            j0 = 2 * u
            j1 = j0 + 1
            pltpu.async_copy(h2_hbm.at[src_v.at[j1]], rows_v1, sem1)
            pltpu.make_async_copy(h2_hbm.at[src_v.at[j0]], rows_v0, sem0).wait()
            pltpu.sync_copy(rows_v0, agg_sh.at[dst_v.at[j0]], add=True)
            jn = lax.min(j0 + 2, HCHUNK - 1)
            pltpu.async_copy(h2_hbm.at[src_v.at[jn]], rows_v0, sem0)
            pltpu.make_async_copy(h2_hbm.at[src_v.at[j1]], rows_v1, sem1).wait()
            pltpu.sync_copy(rows_v1, agg_sh.at[dst_v.at[j1]], add=True)
            return carry

        lax.fori_loop(0, HCHUNK // 2, u_body, 0)
        pltpu.make_async_copy(h2_hbm.at[src_v.at[0]], rows_v0, sem0).wait()

    run_half(0)
    run_half(1)
    plsc.subcore_barrier()

    pltpu.sync_copy(agg_sh.at[pl.ds(row0, ROW_LEN)],
                    out_hbm.at[c, pl.ds(row0, ROW_LEN)])


_ROW_BLK = 2000


def _tc_matmul_body(a0_ref, a1_ref, w_ref, b_ref, o_ref):
    w = w_ref[...]
    acc = lax.dot_general(a0_ref[0], w[:, :DH],
                          (((1,), (1,)), ((), ())),
                          preferred_element_type=jnp.float32)
    acc += lax.dot_general(a1_ref[0], w[:, DH:],
                           (((1,), (1,)), ((), ())),
                           preferred_element_type=jnp.float32)
    o_ref[...] = acc + b_ref[...]


@jax.jit
def kernel(edge_index, h, W, b):
    src = edge_index[0]
    dst = edge_index[1]
    src_t = src.reshape(NS, NCHUNK, CHUNK)
    src2 = jnp.stack([2 * src_t, 2 * src_t + 1])
    dst_t = dst.reshape(NS, NCHUNK, CHUNK)
    h2 = h.reshape(2 * N, DH)
    zeros = jnp.zeros((ROW_LEN, DH), jnp.float32)

    agg2 = _sc_aggregate(src2, dst_t, h2, zeros)

    out = pl.pallas_call(
        _tc_matmul_body,
        grid=(N // _ROW_BLK,),
        in_specs=[
            pl.BlockSpec((1, _ROW_BLK, DH), lambda i: (0, i, 0)),
            pl.BlockSpec((1, _ROW_BLK, DH), lambda i: (1, i, 0)),
            pl.BlockSpec((H, D), lambda i: (0, 0)),
            pl.BlockSpec((1, H), lambda i: (0, 0)),
        ],
        out_specs=pl.BlockSpec((_ROW_BLK, H), lambda i: (i, 0)),
        out_shape=jax.ShapeDtypeStruct((N, H), jnp.float32),
    )(agg2, agg2, W, b.reshape(1, H))
    return out
